# Initial kernel scaffold; baseline (speedup 1.0000x reference)
import jax
import jax.numpy as jnp
from jax.experimental import pallas as pl


def kernel(x, custom_indices, custom_table, regular_table, W, b):
    raise NotImplementedError("write your pallas kernel here")


# speedup vs baseline: 8.5179x; 8.5179x over previous
import jax
import jax.numpy as jnp
from jax import lax
from jax.experimental import pallas as pl
from jax.experimental.pallas import tpu as pltpu
from jax.experimental.pallas import tpu_sc as plsc

D = 64
CUST = 4096
R_TAB = 100000
BLK = 5000
NC, NS = 2, 16
NW = NC * NS
CH = 512
IDX_W = 128
ROWS_T = CUST // NS
N_CI = 2048
CI_T = N_CI // NS


def _matmul_body(ct_ref, w_ref, b_ref, t_ref):
    t_ref[...] = lax.dot_general(
        ct_ref[...], w_ref[...], (((1,), (1,)), ((), ())),
        preferred_element_type=jnp.float32,
        precision=lax.Precision.HIGHEST,
    ) + b_ref[...]


def _patch_body(reg_hbm, t_hbm, ci_hbm, b_hbm, p_hbm,
                rows_v, trows_v, ci_v, b_v, sem):
    c = lax.axis_index("c")
    s = lax.axis_index("s")

    @pl.when(c == 0)
    def _():
        r0 = s * ROWS_T
        pltpu.sync_copy(reg_hbm.at[pl.ds(r0, ROWS_T)], rows_v)
        pltpu.sync_copy(b_hbm, b_v)
        bvals = [b_v[pl.ds(16 * k, 16)] for k in range(4)]

        def addb(r, carry):
            for k in range(4):
                sl = pl.ds(16 * k, 16)
                rows_v[r, sl] = rows_v[r, sl] + bvals[k]
            return carry

        lax.fori_loop(0, ROWS_T, addb, 0)
        pltpu.sync_copy(rows_v, p_hbm.at[pl.ds(r0, ROWS_T)])
        plsc.subcore_barrier()
        pltpu.sync_copy(ci_hbm.at[pl.ds(s * CI_T, CI_T)], ci_v)
        pltpu.async_copy(t_hbm.at[ci_v], trows_v, sem).wait()
        pltpu.async_copy(trows_v, p_hbm.at[ci_v], sem).wait()


def _merge_body(reg_ref, p_ref, b_ref, out_ref):
    i = pl.program_id(0)
    out_ref[...] = reg_ref[...] + b_ref[...]

    @pl.when(i == 0)
    def _():
        out_ref[0:CUST, :] = p_ref[...]


def _gather_body(tab_hbm, x_hbm, out_hbm, idx_v, rows_v, sem):
    c = lax.axis_index("c")
    s = lax.axis_index("s")
    per_w = x_hbm.shape[0] // NW
    iters = per_w // CH
    base = (s * NC + c) * per_w

    def step(g, carry):
        off = base + g * CH
        pltpu.sync_copy(x_hbm.at[pl.ds(off, CH)], idx_v)
        cps = [
            pltpu.async_copy(
                tab_hbm.at[idx_v.at[pl.ds(j * IDX_W, IDX_W)]],
                rows_v.at[pl.ds(j * IDX_W, IDX_W)],
                sem,
            )
            for j in range(CH // IDX_W)
        ]
        for cp in cps:
            cp.wait()
        pltpu.sync_copy(rows_v, out_hbm.at[pl.ds(off, CH)])
        return carry

    lax.fori_loop(0, iters, step, 0)


def kernel(x, custom_indices, custom_table, regular_table, W, b):
    B, L = x.shape
    x_flat = x.reshape(-1).astype(jnp.int32)
    b1 = b.astype(jnp.float32)
    b2 = b1.reshape(1, D)
    ci = custom_indices.reshape(N_CI).astype(jnp.int32)

    t_tab = pl.pallas_call(
        _matmul_body,
        out_shape=jax.ShapeDtypeStruct((CUST, D), jnp.float32),
    )(custom_table, W, b2)

    sc_mesh = plsc.VectorSubcoreMesh(core_axis_name="c", subcore_axis_name="s")
    sc_params = pltpu.CompilerParams(use_tc_tiling_on_sc=False)

    patch = pl.kernel(
        _patch_body,
        out_type=jax.ShapeDtypeStruct((CUST, D), jnp.float32),
        mesh=sc_mesh,
        compiler_params=sc_params,
        scratch_types=[
            pltpu.VMEM((ROWS_T, D), jnp.float32),
            pltpu.VMEM((CI_T, D), jnp.float32),
            pltpu.VMEM((CI_T,), jnp.int32),
            pltpu.VMEM((D,), jnp.float32),
            pltpu.SemaphoreType.DMA,
        ],
    )
    p_tab = patch(regular_table, t_tab, ci, b1)

    merged = pl.pallas_call(
        _merge_body,
        grid=(R_TAB // BLK,),
        in_specs=[
            pl.BlockSpec((BLK, D), lambda i: (i, 0)),
            pl.BlockSpec((CUST, D), lambda i: (0, 0)),
            pl.BlockSpec((1, D), lambda i: (0, 0)),
        ],
        out_specs=pl.BlockSpec((BLK, D), lambda i: (i, 0)),
        out_shape=jax.ShapeDtypeStruct((R_TAB, D), jnp.float32),
    )(regular_table, p_tab, b2)

    gather = pl.kernel(
        _gather_body,
        out_type=jax.ShapeDtypeStruct((B * L, D), jnp.float32),
        mesh=sc_mesh,
        compiler_params=sc_params,
        scratch_types=[
            pltpu.VMEM((CH,), jnp.int32),
            pltpu.VMEM((CH, D), jnp.float32),
            pltpu.SemaphoreType.DMA,
        ],
    )
    out_flat = gather(merged, x_flat)
    return out_flat.reshape(B, L, D)
